# baseline (speedup 1.0000x reference)
import functools

import jax
import jax.numpy as jnp
from jax import lax
from jax.experimental import pallas as pl
from jax.experimental.pallas import tpu as pltpu
from jax.experimental.pallas import tpu_sc as plsc

_VOCAB = 100000
_MAXLEN = 200
_EMBED = 64
_BATCH = 1024

_NW = 32
_CHUNK = 100
_ROWS_PER_W = (_BATCH * _MAXLEN) // _NW
_CHUNKS_PER_W = _ROWS_PER_W // _CHUNK
_GROUP = 8
_GROUPS_PER_W = _CHUNKS_PER_W // _GROUP


def _make_kernel():
    mesh = plsc.VectorSubcoreMesh(core_axis_name="c", subcore_axis_name="s")

    @functools.partial(
        pl.kernel,
        mesh=mesh,
        out_type=jax.ShapeDtypeStruct(
            (_NW * _GROUPS_PER_W, _GROUP * _CHUNK, _EMBED), jnp.float32
        ),
        scratch_types=[
            pltpu.VMEM((_CHUNKS_PER_W, _CHUNK), jnp.int32),
            pltpu.VMEM((_MAXLEN, _EMBED), jnp.float32),
            pltpu.VMEM((_GROUP * _CHUNK, _EMBED), jnp.float32),
            pltpu.VMEM((_GROUP * _CHUNK, _EMBED), jnp.float32),
            [pltpu.SemaphoreType.DMA] * _GROUP,
            pltpu.SemaphoreType.DMA,
            pltpu.SemaphoreType.DMA,
        ],
        compiler_params=pltpu.CompilerParams(use_tc_tiling_on_sc=False),
    )
    def emb_kernel(x_hbm, tok_hbm, pos_hbm, out_hbm,
                   idx_v, pos_v, rows_a, rows_b, gsem, wsa, wsb):
        cid = lax.axis_index("c")
        sid = lax.axis_index("s")
        wid = sid * 2 + cid
        base = wid * _CHUNKS_PER_W
        gbase = wid * _GROUPS_PER_W

        pltpu.sync_copy(pos_hbm, pos_v)
        pltpu.sync_copy(x_hbm.at[pl.ds(base, _CHUNKS_PER_W)], idx_v)

        def fill(buf, j0):
            gh = []
            for t in range(_GROUP):
                gh.append(
                    pltpu.async_copy(
                        tok_hbm.at[idx_v.at[j0 + t]],
                        buf.at[pl.ds(_CHUNK * t, _CHUNK)],
                        gsem[t],
                    )
                )
            for t in range(_GROUP):
                gh[t].wait()
                poff = (t & 1) * _CHUNK

                def add_row(r, c2):
                    for cc in range(_EMBED // 16):
                        sl = pl.ds(cc * 16, 16)
                        buf[_CHUNK * t + r, sl] = (
                            buf[_CHUNK * t + r, sl] + pos_v[poff + r, sl]
                        )
                    return c2

                lax.fori_loop(0, _CHUNK, add_row, 0)

        def pair_body(ii, carry):
            j0 = ii * 2 * _GROUP
            fill(rows_a, j0)
            wha = pltpu.async_copy(rows_a, out_hbm.at[gbase + 2 * ii], wsa)
            fill(rows_b, j0 + _GROUP)
            whb = pltpu.async_copy(rows_b, out_hbm.at[gbase + 2 * ii + 1], wsb)
            wha.wait()
            whb.wait()
            return carry

        lax.fori_loop(0, _GROUPS_PER_W // 2, pair_body, 0)

    return emb_kernel


_EMB_KERNEL = _make_kernel()


@jax.jit
def kernel(x, tok_table, pos_table):
    b, maxlen = x.shape
    x2d = x.reshape(-1).astype(jnp.int32).reshape(_NW * _CHUNKS_PER_W, _CHUNK)
    out = _EMB_KERNEL(x2d, tok_table, pos_table)
    return out.reshape(b, maxlen, _EMBED)
